# baseline (device time: 9417 ns/iter reference)
import jax
import jax.numpy as jnp
from jax import lax
from jax.experimental import pallas as pl
from jax.experimental.pallas import tpu as pltpu

X_SIZE = 2


def kernel(x):
    m_per, n = x.shape
    half = n // X_SIZE
    rows_c = m_per // 2

    def body(
        x_ref,
        out_ref,
        send_q0,
        send_q1,
        recv_q0,
        recv_q1,
        send_scale,
        recv_scale,
        send_sems,
        recv_sems,
    ):
        my_x = lax.axis_index("x")
        my_y = lax.axis_index("y")
        my_z = lax.axis_index("z")
        peer = (1 - my_x, my_y, my_z)

        block = x_ref[:, pl.ds((1 - my_x) * half, half)]
        absmax = jnp.maximum(
            jnp.max(jnp.abs(block), axis=0, keepdims=True), 1e-30
        )
        send_scale[:, :] = absmax * (1.0 / 127.0)
        rs = 127.0 / absmax
        send_q0[:, :] = jnp.round(block[:rows_c, :] * rs).astype(jnp.int8)

        barrier_sem = pltpu.get_barrier_semaphore()
        pl.semaphore_signal(
            barrier_sem, inc=1, device_id=peer,
            device_id_type=pl.DeviceIdType.MESH,
        )
        pl.semaphore_wait(barrier_sem, 1)

        def remote(src, dst, i):
            return pltpu.make_async_remote_copy(
                src_ref=src,
                dst_ref=dst,
                send_sem=send_sems.at[i],
                recv_sem=recv_sems.at[i],
                device_id=peer,
                device_id_type=pl.DeviceIdType.MESH,
            )

        rdma_s = remote(send_scale, recv_scale, 0)
        rdma_q0 = remote(send_q0, recv_q0, 1)
        rdma_s.start()
        rdma_q0.start()

        send_q1[:, :] = jnp.round(block[rows_c:, :] * rs).astype(jnp.int8)
        rdma_q1 = remote(send_q1, recv_q1, 2)
        rdma_q1.start()

        out_ref[pl.ds(my_x * m_per, m_per), :] = x_ref[:, pl.ds(my_x * half, half)]

        other_base = (1 - my_x) * m_per
        rdma_s.wait()
        rdma_q0.wait()
        out_ref[pl.ds(other_base, rows_c), :] = (
            recv_q0[:, :].astype(jnp.float32) * recv_scale[:, :]
        )
        rdma_q1.wait()
        out_ref[pl.ds(other_base + rows_c, rows_c), :] = (
            recv_q1[:, :].astype(jnp.float32) * recv_scale[:, :]
        )

    return pl.pallas_call(
        body,
        out_shape=jax.ShapeDtypeStruct((X_SIZE * m_per, half), x.dtype),
        in_specs=[pl.BlockSpec(memory_space=pltpu.VMEM)],
        out_specs=pl.BlockSpec(memory_space=pltpu.VMEM),
        scratch_shapes=[
            pltpu.VMEM((rows_c, half), jnp.int8),
            pltpu.VMEM((rows_c, half), jnp.int8),
            pltpu.VMEM((rows_c, half), jnp.int8),
            pltpu.VMEM((rows_c, half), jnp.int8),
            pltpu.VMEM((1, half), jnp.float32),
            pltpu.VMEM((1, half), jnp.float32),
            pltpu.SemaphoreType.DMA((3,)),
            pltpu.SemaphoreType.DMA((3,)),
        ],
        compiler_params=pltpu.CompilerParams(collective_id=0),
    )(x)
